# baseline (speedup 1.0000x reference)
import functools

import jax
import jax.numpy as jnp
import numpy as np
from jax import lax
from jax.experimental import pallas as pl
from jax.experimental.pallas import tpu as pltpu
from jax.experimental.pallas import tpu_sc as plsc

HIDDEN = 64


def _make_gather(N, D, C, NB):
    info = plsc.get_sparse_core_info()
    NC, NS = info.num_cores, info.num_subcores
    NW = NC * NS
    b_per_w = N // NW
    n_chunks = b_per_w // C
    n_groups = n_chunks // NB
    assert C == 256
    assert N % NW == 0 and b_per_w % C == 0 and n_chunks % NB == 0
    assert n_groups >= 2
    mesh = plsc.VectorSubcoreMesh(core_axis_name="c", subcore_axis_name="s")

    @functools.partial(
        pl.kernel,
        mesh=mesh,
        out_type=jax.ShapeDtypeStruct((N // 2, 2 * D), jnp.float32),
        scratch_types=[
            pltpu.VMEM((b_per_w,), jnp.int32),
            pltpu.VMEM((NB * C, D), jnp.float32),
        ]
        + [pltpu.SemaphoreType.DMA] * (2 * NB),
        compiler_params=pltpu.CompilerParams(use_tc_tiling_on_sc=False),
    )
    def k(idx_hbm, table_hbm, out_hbm, idx_v, rows_v, *sems):
        gsems, wsems = sems[:NB], sems[NB:]
        wid = lax.axis_index("s") * NC + lax.axis_index("c")
        base = wid * b_per_w
        pltpu.sync_copy(idx_hbm.at[pl.ds(base, b_per_w)], idx_v)

        def fire_gather(i, b):
            pltpu.async_copy(
                table_hbm.at[idx_v.at[pl.ds(i * C, C)]],
                rows_v.at[pl.ds(b * C, C)],
                gsems[b],
            )

        def fire_write(i, b):
            line0 = (base + i * C) // 2
            h = C // 2
            pltpu.async_copy(
                rows_v.at[pl.ds(b * C, h)],
                out_hbm.at[pl.ds(line0, h), pl.ds(0, D)],
                wsems[b],
            )
            pltpu.async_copy(
                rows_v.at[pl.ds(b * C + h, h)],
                out_hbm.at[pl.ds(line0, h), pl.ds(D, D)],
                wsems[b],
            )

        def wait(sem, b):
            pltpu.make_async_copy(
                table_hbm.at[pl.ds(0, C)], rows_v.at[pl.ds(b * C, C)], sem
            ).wait()

        for b in range(NB):
            fire_gather(b, b)

        def body(g, carry):
            i0 = g * NB
            for b in range(NB):
                wait(gsems[b], b)
                fire_write(i0 + b, b)
            for b in range(NB):
                wait(wsems[b], b)
                fire_gather(i0 + NB + b, b)
            return carry

        lax.fori_loop(0, n_groups - 1, body, 0)

        i0 = (n_groups - 1) * NB
        for b in range(NB):
            wait(gsems[b], b)
            fire_write(i0 + b, b)
        for b in range(NB):
            wait(wsems[b], b)

    return k


def _tc_relayout(lines3, L, B, D):
    npt = B // 256

    def body(x_ref, o_ref):
        t = x_ref[0].T
        for pt in range(npt):
            blk = t[:, pt * 128 : (pt + 1) * 128]
            o_ref[0, :, 2 * pt, :, :] = blk[0:D].reshape(D // 8, 8, 128)
            o_ref[0, :, 2 * pt + 1, :, :] = blk[D : 2 * D].reshape(D // 8, 8, 128)

    return pl.pallas_call(
        body,
        grid=(L,),
        in_specs=[
            pl.BlockSpec((1, B // 2, 128), lambda l: (l, 0, 0)),
        ],
        out_specs=pl.BlockSpec(
            (1, D // 8, B // 128, 8, 128), lambda l: (l, 0, 0, 0, 0)
        ),
        out_shape=jax.ShapeDtypeStruct(
            (L, D // 8, B // 128, 8, 128), jnp.float32
        ),
    )(lines3)


def kernel(x, table):
    B, L = x.shape
    N = B * L
    idx = x.T.reshape(N).astype(jnp.int32)
    lines = _make_gather(N, HIDDEN, 256, 5)(idx, table)
    lines3 = lines.reshape(L, B // 2, 128)
    out5 = _tc_relayout(lines3, L, B, HIDDEN)
    return out5.transpose((2, 4, 0, 1, 3)).reshape(B, L, HIDDEN)


# speedup vs baseline: 1.1216x; 1.1216x over previous
import functools

import jax
import jax.numpy as jnp
from jax import lax
from jax.experimental import pallas as pl
from jax.experimental.pallas import tpu as pltpu
from jax.experimental.pallas import tpu_sc as plsc

HIDDEN = 64


def _make_gather(N, D, C, NB):
    info = plsc.get_sparse_core_info()
    NC, NS = info.num_cores, info.num_subcores
    NW = NC * NS
    b_per_w = N // NW
    n_chunks = b_per_w // C
    n_groups = n_chunks // NB
    assert N % NW == 0 and b_per_w % C == 0 and n_chunks % NB == 0
    assert n_groups >= 2
    mesh = plsc.VectorSubcoreMesh(core_axis_name="c", subcore_axis_name="s")

    @functools.partial(
        pl.kernel,
        mesh=mesh,
        out_type=jax.ShapeDtypeStruct((N, 128), jnp.float32),
        scratch_types=[
            pltpu.VMEM((b_per_w,), jnp.int32),
            pltpu.VMEM((NB * C, D), jnp.float32),
        ]
        + [pltpu.SemaphoreType.DMA] * (2 * NB),
        compiler_params=pltpu.CompilerParams(use_tc_tiling_on_sc=False),
    )
    def k(idx_hbm, table_hbm, out_hbm, idx_v, rows_v, *sems):
        gsems, wsems = sems[:NB], sems[NB:]
        wid = lax.axis_index("s") * NC + lax.axis_index("c")
        base = wid * b_per_w
        pltpu.sync_copy(idx_hbm.at[pl.ds(base, b_per_w)], idx_v)

        def fire_gather(i, b):
            pltpu.async_copy(
                table_hbm.at[idx_v.at[pl.ds(i * C, C)]],
                rows_v.at[pl.ds(b * C, C)],
                gsems[b],
            )

        def fire_write(i, b):
            pltpu.async_copy(
                rows_v.at[pl.ds(b * C, C)],
                out_hbm.at[pl.ds(base + i * C, C), pl.ds(0, D)],
                wsems[b],
            )

        def wait(sem, b):
            pltpu.make_async_copy(
                table_hbm.at[pl.ds(0, C)], rows_v.at[pl.ds(b * C, C)], sem
            ).wait()

        for b in range(NB):
            fire_gather(b, b)

        def body(g, carry):
            i0 = g * NB
            for b in range(NB):
                wait(gsems[b], b)
                fire_write(i0 + b, b)
            for b in range(NB):
                wait(wsems[b], b)
                fire_gather(i0 + NB + b, b)
            return carry

        lax.fori_loop(0, n_groups - 1, body, 0)

        i0 = (n_groups - 1) * NB
        for b in range(NB):
            wait(gsems[b], b)
            fire_write(i0 + b, b)
        for b in range(NB):
            wait(wsems[b], b)

    return k


def kernel(x, table):
    B, L = x.shape
    N = B * L
    idx = x.reshape(N).astype(jnp.int32)
    out = _make_gather(N, HIDDEN, 256, 5)(idx, table)
    return lax.slice(out, (0, 0), (N, HIDDEN)).reshape(B, L, HIDDEN)
